# Initial kernel scaffold; baseline (speedup 1.0000x reference)
import functools

import jax
import jax.numpy as jnp
from jax import lax
from jax.experimental import pallas as pl
from jax.experimental.pallas import tpu as pltpu
from jax.experimental.pallas import tpu_sc as plsc

N = 10000
E = 320000
D = 128
DA = 144
NC, NS = 2, 16
NW = NC * NS
EPW = E // NW
CH = 80
NCHUNK = EPW // CH
RPS = N // NS

BN = 1000


def _d1_body(x_ref, w_ref, b_ref, out_ref):
    h = jnp.dot(x_ref[...], w_ref[...], preferred_element_type=jnp.float32)
    h = jnp.maximum(h + b_ref[...], 0.0)
    col = lax.broadcasted_iota(jnp.int32, (h.shape[0], DA - D), 1)
    pad = jnp.where(col == 0, 1.0, 0.0)
    out_ref[...] = jnp.concatenate([h, pad], axis=1)


def _d1(x, w, b):
    return pl.pallas_call(
        _d1_body,
        grid=(N // BN,),
        in_specs=[
            pl.BlockSpec((BN, D), lambda i: (i, 0)),
            pl.BlockSpec((D, D), lambda i: (0, 0)),
            pl.BlockSpec((1, D), lambda i: (0, 0)),
        ],
        out_specs=pl.BlockSpec((BN, DA), lambda i: (i, 0)),
        out_shape=jax.ShapeDtypeStruct((N, DA), jnp.float32),
    )(x, w, b)


_MESH = plsc.VectorSubcoreMesh(
    core_axis_name="c", subcore_axis_name="s", num_cores=NC, num_subcores=NS)


@functools.partial(
    pl.kernel,
    out_type=jax.ShapeDtypeStruct((NC, N, DA), jnp.float32),
    mesh=_MESH,
    scratch_types=[
        pltpu.VMEM((NCHUNK, CH), jnp.int32),
        pltpu.VMEM((NCHUNK, CH), jnp.int32),
        pltpu.VMEM((CH, DA), jnp.float32),
        pltpu.VMEM((CH, DA), jnp.float32),
        pltpu.VMEM_SHARED((N, DA), jnp.float32),
        pltpu.SemaphoreType.DMA,
        pltpu.SemaphoreType.DMA,
    ],
)
def _sc_agg(h_hbm, src_hbm, dst_hbm, zeros_hbm, out_hbm,
            src_v, dst_v, rows_a, rows_b, agg_sh, sem_a, sem_b):
    cid = lax.axis_index("c")
    sid = lax.axis_index("s")
    wid = cid * NS + sid

    pltpu.sync_copy(zeros_hbm, agg_sh.at[pl.ds(sid * RPS, RPS)])
    pltpu.sync_copy(src_hbm.at[wid], src_v)
    pltpu.sync_copy(dst_hbm.at[wid], dst_v)
    plsc.subcore_barrier()

    def _gather(c, buf, sem):
        pltpu.async_copy(h_hbm.at[src_v.at[c]], buf, sem)

    def _wait(c, buf, sem):
        pltpu.make_async_copy(h_hbm.at[src_v.at[c]], buf, sem).wait()

    _gather(0, rows_a, sem_a)

    def _pair(p, carry):
        c = 2 * p
        _gather(c + 1, rows_b, sem_b)
        _wait(c, rows_a, sem_a)
        pltpu.sync_copy(rows_a, agg_sh.at[dst_v.at[c]], add=True)
        _gather(c + 2, rows_a, sem_a)
        _wait(c + 1, rows_b, sem_b)
        pltpu.sync_copy(rows_b, agg_sh.at[dst_v.at[c + 1]], add=True)
        return carry

    lax.fori_loop(0, (NCHUNK - 1) // 2, _pair, 0)
    _wait(NCHUNK - 1, rows_a, sem_a)
    pltpu.sync_copy(rows_a, agg_sh.at[dst_v.at[NCHUNK - 1]], add=True)

    plsc.subcore_barrier()
    pltpu.sync_copy(agg_sh.at[pl.ds(sid * RPS, RPS)],
                    out_hbm.at[cid, pl.ds(sid * RPS, RPS)])


def _out_body(part_ref, h_ref, ws_ref, bs_ref, wn_ref, w2_ref, b2_ref, o_ref):
    p = part_ref[0] + part_ref[1]
    agg = p[:, :D]
    deg = jnp.maximum(p[:, D:D + 1], 1.0)
    h_neigh = agg / deg
    h = h_ref[:, :D]
    h2 = jnp.dot(h, ws_ref[...], preferred_element_type=jnp.float32)
    h2 = h2 + jnp.dot(h_neigh, wn_ref[...], preferred_element_type=jnp.float32)
    h2 = jnp.maximum(h2 + bs_ref[...], 0.0)
    o = jnp.dot(h2, w2_ref[...], preferred_element_type=jnp.float32)
    o_ref[...] = jnp.maximum(o + b2_ref[...], 0.0)


def _out(part, h_aug, ws, bs, wn, w2, b2):
    return pl.pallas_call(
        _out_body,
        grid=(N // BN,),
        in_specs=[
            pl.BlockSpec((NC, BN, DA), lambda i: (0, i, 0)),
            pl.BlockSpec((BN, DA), lambda i: (i, 0)),
            pl.BlockSpec((D, D), lambda i: (0, 0)),
            pl.BlockSpec((1, D), lambda i: (0, 0)),
            pl.BlockSpec((D, D), lambda i: (0, 0)),
            pl.BlockSpec((D, D), lambda i: (0, 0)),
            pl.BlockSpec((1, D), lambda i: (0, 0)),
        ],
        out_specs=pl.BlockSpec((BN, D), lambda i: (i, 0)),
        out_shape=jax.ShapeDtypeStruct((N, D), jnp.float32),
    )(part, h_aug, ws, bs, wn, w2, b2)


def kernel(x, edge_index, W_d1, b_d1, W_self, b_self, W_neigh, W_d2, b_d2):
    h_aug = _d1(x, W_d1, b_d1.reshape(1, D))
    src = edge_index[0].reshape(NW, NCHUNK, CH)
    dst = edge_index[1].reshape(NW, NCHUNK, CH)
    zeros = jnp.zeros((RPS, DA), jnp.float32)
    part = _sc_agg(h_aug, src, dst, zeros)
    return _out(part, h_aug, W_self, b_self.reshape(1, D), W_neigh,
                W_d2, b_d2.reshape(1, D))


# speedup vs baseline: 13.1089x; 13.1089x over previous
import functools

import jax
import jax.numpy as jnp
from jax import lax
from jax.experimental import pallas as pl
from jax.experimental.pallas import tpu as pltpu
from jax.experimental.pallas import tpu_sc as plsc

N = 10000
E = 320000
D = 128
NC, NS = 2, 16
NW = NC * NS
EPW = E // NW
CH = 80
NCHUNK = EPW // CH
NCP = NCHUNK + 3
NQ = (NCHUNK - 1) // 4
NP = 10240
RPS = NP // NS
L = 16

BN = 1024


def _d1_body(x_ref, w_ref, b_ref, out_ref):
    h = jnp.dot(x_ref[...], w_ref[...], preferred_element_type=jnp.float32)
    out_ref[...] = jnp.maximum(h + b_ref[...], 0.0)


def _d1(x, w, b):
    return pl.pallas_call(
        _d1_body,
        grid=(NP // BN,),
        in_specs=[
            pl.BlockSpec((BN, D), lambda i: (i, 0)),
            pl.BlockSpec((D, D), lambda i: (0, 0)),
            pl.BlockSpec((1, D), lambda i: (0, 0)),
        ],
        out_specs=pl.BlockSpec((BN, D), lambda i: (i, 0)),
        out_shape=jax.ShapeDtypeStruct((N, D), jnp.float32),
    )(x, w, b)


_MESH = plsc.VectorSubcoreMesh(
    core_axis_name="c", subcore_axis_name="s", num_cores=NC, num_subcores=NS)


@functools.partial(
    pl.kernel,
    out_type=(jax.ShapeDtypeStruct((NC, NP, D), jnp.float32),
              jax.ShapeDtypeStruct((NW, N), jnp.float32)),
    mesh=_MESH,
    compiler_params=pltpu.CompilerParams(needs_layout_passes=False),
    scratch_types=[
        pltpu.VMEM((2, 2, CH), jnp.int32),
        pltpu.VMEM((2, 2, CH), jnp.int32),
        pltpu.VMEM((N,), jnp.float32),
        pltpu.VMEM((CH, D), jnp.float32),
        pltpu.VMEM((CH, D), jnp.float32),
        pltpu.VMEM_SHARED((NP, D), jnp.float32),
        pltpu.SemaphoreType.DMA,
        pltpu.SemaphoreType.DMA,
        pltpu.SemaphoreType.DMA,
        pltpu.SemaphoreType.DMA,
    ],
)
def _sc_agg(h_hbm, idx_hbm, zeros_hbm, out_hbm, deg_hbm,
            pib_a, pib_b, hist_v, rows_a, rows_b, agg_sh,
            isem_a, isem_b, sem_a, sem_b):
    cid = lax.axis_index("c")
    sid = lax.axis_index("s")
    wid = cid * NS + sid

    pltpu.sync_copy(zeros_hbm, agg_sh.at[pl.ds(sid * RPS, RPS)])

    def _idx_load(p, pib, isem):
        pltpu.async_copy(idx_hbm.at[wid, pl.ds(2 * p, 2)], pib, isem)

    def _idx_wait(pib, isem):
        pltpu.make_async_copy(idx_hbm.at[wid, pl.ds(0, 2)], pib, isem).wait()

    def _gather(idx_row, buf, sem):
        pltpu.async_copy(h_hbm.at[idx_row], buf, sem)

    def _gwait(buf, sem):
        pltpu.make_async_copy(h_hbm.at[pib_a.at[0, 0]], buf, sem).wait()

    _idx_load(0, pib_a, isem_a)
    _idx_wait(pib_a, isem_a)
    _idx_load(1, pib_b, isem_b)
    _gather(pib_a.at[0, 0], rows_a, sem_a)

    zeros16 = jnp.zeros((L,), jnp.float32)

    def _zero_hist(j, carry):
        hist_v[pl.ds(j * L, L)] = zeros16
        return carry

    lax.fori_loop(0, N // L, _zero_hist, 0)
    plsc.subcore_barrier()

    ones = jnp.ones((L,), jnp.float32)

    def _hist(pib, k):
        for j in range(CH // L):
            d16 = pib[k, 1, pl.ds(j * L, L)]
            plsc.addupdate_scatter(hist_v, [d16], ones)

    def _scat(buf, idx_row):
        pltpu.sync_copy(buf, agg_sh.at[idx_row], add=True)

    def _quad(q, carry):
        _gather(pib_a.at[1, 0], rows_b, sem_b)
        _gwait(rows_a, sem_a)
        _scat(rows_a, pib_a.at[0, 1])
        _hist(pib_a, 0)
        _idx_wait(pib_b, isem_b)
        _gather(pib_b.at[0, 0], rows_a, sem_a)
        _gwait(rows_b, sem_b)
        _scat(rows_b, pib_a.at[1, 1])
        _hist(pib_a, 1)
        _idx_load(2 * q + 2, pib_a, isem_a)
        _gather(pib_b.at[1, 0], rows_b, sem_b)
        _gwait(rows_a, sem_a)
        _scat(rows_a, pib_b.at[0, 1])
        _hist(pib_b, 0)
        _idx_wait(pib_a, isem_a)
        _gather(pib_a.at[0, 0], rows_a, sem_a)
        _gwait(rows_b, sem_b)
        _scat(rows_b, pib_b.at[1, 1])
        _hist(pib_b, 1)
        _idx_load(2 * q + 3, pib_b, isem_b)
        return carry

    lax.fori_loop(0, NQ, _quad, 0)
    _gwait(rows_a, sem_a)
    _scat(rows_a, pib_a.at[0, 1])
    _hist(pib_a, 0)
    _idx_wait(pib_b, isem_b)

    pltpu.sync_copy(hist_v, deg_hbm.at[wid])
    plsc.subcore_barrier()
    pltpu.sync_copy(agg_sh.at[pl.ds(sid * RPS, RPS)],
                    out_hbm.at[cid, pl.ds(sid * RPS, RPS)])


def _out_body(part_ref, deg_ref, h_ref, ws_ref, bs_ref, wn_ref, w2_ref,
              b2_ref, o_ref):
    agg = part_ref[0] + part_ref[1]
    degt = jnp.transpose(deg_ref[...])
    deg = jnp.maximum(jnp.sum(degt, axis=1, keepdims=True), 1.0)
    h_neigh = agg / deg
    h = h_ref[...]
    h2 = jnp.dot(h, ws_ref[...], preferred_element_type=jnp.float32)
    h2 = h2 + jnp.dot(h_neigh, wn_ref[...], preferred_element_type=jnp.float32)
    h2 = jnp.maximum(h2 + bs_ref[...], 0.0)
    o = jnp.dot(h2, w2_ref[...], preferred_element_type=jnp.float32)
    o_ref[...] = jnp.maximum(o + b2_ref[...], 0.0)


def _out(part, deg, h, ws, bs, wn, w2, b2):
    return pl.pallas_call(
        _out_body,
        grid=(NP // BN,),
        in_specs=[
            pl.BlockSpec((NC, BN, D), lambda i: (0, i, 0)),
            pl.BlockSpec((NW, BN), lambda i: (0, i)),
            pl.BlockSpec((BN, D), lambda i: (i, 0)),
            pl.BlockSpec((D, D), lambda i: (0, 0)),
            pl.BlockSpec((1, D), lambda i: (0, 0)),
            pl.BlockSpec((D, D), lambda i: (0, 0)),
            pl.BlockSpec((D, D), lambda i: (0, 0)),
            pl.BlockSpec((1, D), lambda i: (0, 0)),
        ],
        out_specs=pl.BlockSpec((BN, D), lambda i: (i, 0)),
        out_shape=jax.ShapeDtypeStruct((N, D), jnp.float32),
    )(part, deg, h, ws, bs, wn, w2, b2)


def kernel(x, edge_index, W_d1, b_d1, W_self, b_self, W_neigh, W_d2, b_d2):
    h = _d1(x, W_d1, b_d1.reshape(1, D))
    idx = jnp.transpose(edge_index.reshape(2, NW, NCHUNK, CH), (1, 2, 0, 3))
    idx = jnp.pad(idx, ((0, 0), (0, NCP - NCHUNK), (0, 0), (0, 0)))
    zeros = jnp.zeros((RPS, D), jnp.float32)
    part, deg = _sc_agg(h, idx, zeros)
    return _out(part, deg, h, W_self,
                b_self.reshape(1, D), W_neigh, W_d2, b_d2.reshape(1, D))


# speedup vs baseline: 13.8301x; 1.0550x over previous
import functools

import jax
import jax.numpy as jnp
from jax import lax
from jax.experimental import pallas as pl
from jax.experimental.pallas import tpu as pltpu
from jax.experimental.pallas import tpu_sc as plsc

N = 10000
E = 320000
D = 128
NC, NS = 2, 16
NW = NC * NS
EPW = E // NW
CH = 80
NCHUNK = EPW // CH
NTRIO = NCHUNK // 3
NP = 10240
RPS = NP // NS
L = 16

BN = 1024


def _d1_body(x_ref, w_ref, b_ref, out_ref):
    h = jnp.dot(x_ref[...], w_ref[...], preferred_element_type=jnp.float32)
    out_ref[...] = jnp.maximum(h + b_ref[...], 0.0)


def _d1(x, w, b):
    return pl.pallas_call(
        _d1_body,
        grid=(NP // BN,),
        in_specs=[
            pl.BlockSpec((BN, D), lambda i: (i, 0)),
            pl.BlockSpec((D, D), lambda i: (0, 0)),
            pl.BlockSpec((1, D), lambda i: (0, 0)),
        ],
        out_specs=pl.BlockSpec((BN, D), lambda i: (i, 0)),
        out_shape=jax.ShapeDtypeStruct((N, D), jnp.float32),
    )(x, w, b)


_MESH = plsc.VectorSubcoreMesh(
    core_axis_name="c", subcore_axis_name="s", num_cores=NC, num_subcores=NS)


@functools.partial(
    pl.kernel,
    out_type=(jax.ShapeDtypeStruct((NC, NP, D), jnp.float32),
              jax.ShapeDtypeStruct((NW, N), jnp.float32)),
    mesh=_MESH,
    compiler_params=pltpu.CompilerParams(needs_layout_passes=False),
    scratch_types=[
        pltpu.VMEM((3, 2, CH), jnp.int32),
        pltpu.VMEM((3, 2, CH), jnp.int32),
        pltpu.VMEM((N,), jnp.float32),
        pltpu.VMEM((CH, D), jnp.float32),
        pltpu.VMEM((CH, D), jnp.float32),
        pltpu.VMEM((CH, D), jnp.float32),
        pltpu.VMEM_SHARED((NP, D), jnp.float32),
        pltpu.SemaphoreType.DMA,
        pltpu.SemaphoreType.DMA,
        pltpu.SemaphoreType.DMA,
        pltpu.SemaphoreType.DMA,
        pltpu.SemaphoreType.DMA,
    ],
)
def _sc_agg(h_hbm, idx_hbm, zeros_hbm, out_hbm, deg_hbm,
            tib_a, tib_b, hist_v, r0, r1, r2, agg_sh,
            g0, g1, g2, isem_a, isem_b):
    cid = lax.axis_index("c")
    sid = lax.axis_index("s")
    wid = cid * NS + sid
    rbufs = (r0, r1, r2)
    gsems = (g0, g1, g2)

    pltpu.sync_copy(zeros_hbm, agg_sh.at[pl.ds(sid * RPS, RPS)])

    def _trio_load(t, tib, isem):
        pltpu.async_copy(idx_hbm.at[wid, pl.ds(3 * t, 3)], tib, isem)

    def _trio_wait(tib, isem):
        pltpu.make_async_copy(idx_hbm.at[wid, pl.ds(0, 3)], tib, isem).wait()

    def _gather(idx_row, buf, sem):
        pltpu.async_copy(h_hbm.at[idx_row], buf, sem)

    def _gwait(buf, sem):
        pltpu.make_async_copy(h_hbm.at[tib_a.at[0, 0]], buf, sem).wait()

    def _scat(buf, idx_row):
        pltpu.sync_copy(buf, agg_sh.at[idx_row], add=True)

    ones = jnp.ones((L,), jnp.float32)

    def _hist(tib, k):
        for j in range(CH // L):
            d16 = tib[k, 1, pl.ds(j * L, L)]
            plsc.addupdate_scatter(hist_v, [d16], ones)

    _trio_load(0, tib_a, isem_a)
    _trio_wait(tib_a, isem_a)
    _trio_load(1, tib_b, isem_b)
    for k in range(3):
        _gather(tib_a.at[k, 0], rbufs[k], gsems[k])

    zeros16 = jnp.zeros((L,), jnp.float32)

    def _zero_hist(j, carry):
        hist_v[pl.ds(j * L, L)] = zeros16
        return carry

    lax.fori_loop(0, N // L, _zero_hist, 0)
    plsc.subcore_barrier()

    def _half(t, cur, nxt, isem_cur, isem_nxt):
        _trio_wait(nxt, isem_nxt)
        for k in range(3):
            _gwait(rbufs[k], gsems[k])
            _scat(rbufs[k], cur.at[k, 1])
            _gather(nxt.at[k, 0], rbufs[k], gsems[k])
            _hist(cur, k)
        _trio_load(jnp.minimum(t + 2, NTRIO - 1), cur, isem_cur)

    def _pair(i, carry):
        t = 2 * i
        _half(t, tib_a, tib_b, isem_a, isem_b)
        _half(t + 1, tib_b, tib_a, isem_b, isem_a)
        return carry

    lax.fori_loop(0, NTRIO // 2, _pair, 0)
    _trio_wait(tib_b, isem_b)
    pltpu.async_copy(idx_hbm.at[wid, pl.ds(NTRIO * 3, 2)],
                     tib_b.at[pl.ds(0, 2)], isem_b)
    pltpu.make_async_copy(idx_hbm.at[wid, pl.ds(0, 2)],
                          tib_b.at[pl.ds(0, 2)], isem_b).wait()
    for k in range(3):
        _gwait(rbufs[k], gsems[k])
        _scat(rbufs[k], tib_a.at[k, 1])
        if k < 2:
            _gather(tib_b.at[k, 0], rbufs[k], gsems[k])
        _hist(tib_a, k)
    for k in range(2):
        _gwait(rbufs[k], gsems[k])
        _scat(rbufs[k], tib_b.at[k, 1])
        _hist(tib_b, k)

    pltpu.sync_copy(hist_v, deg_hbm.at[wid])
    plsc.subcore_barrier()
    pltpu.sync_copy(agg_sh.at[pl.ds(sid * RPS, RPS)],
                    out_hbm.at[cid, pl.ds(sid * RPS, RPS)])


def _out_body(part_ref, deg_ref, h_ref, ws_ref, bs_ref, wn_ref, w2_ref,
              b2_ref, o_ref):
    agg = part_ref[0] + part_ref[1]
    degt = jnp.transpose(deg_ref[...])
    deg = jnp.maximum(jnp.sum(degt, axis=1, keepdims=True), 1.0)
    h_neigh = agg / deg
    h = h_ref[...]
    h2 = jnp.dot(h, ws_ref[...], preferred_element_type=jnp.float32)
    h2 = h2 + jnp.dot(h_neigh, wn_ref[...], preferred_element_type=jnp.float32)
    h2 = jnp.maximum(h2 + bs_ref[...], 0.0)
    o = jnp.dot(h2, w2_ref[...], preferred_element_type=jnp.float32)
    o_ref[...] = jnp.maximum(o + b2_ref[...], 0.0)


def _out(part, deg, h, ws, bs, wn, w2, b2):
    return pl.pallas_call(
        _out_body,
        grid=(NP // BN,),
        in_specs=[
            pl.BlockSpec((NC, BN, D), lambda i: (0, i, 0)),
            pl.BlockSpec((NW, BN), lambda i: (0, i)),
            pl.BlockSpec((BN, D), lambda i: (i, 0)),
            pl.BlockSpec((D, D), lambda i: (0, 0)),
            pl.BlockSpec((1, D), lambda i: (0, 0)),
            pl.BlockSpec((D, D), lambda i: (0, 0)),
            pl.BlockSpec((D, D), lambda i: (0, 0)),
            pl.BlockSpec((1, D), lambda i: (0, 0)),
        ],
        out_specs=pl.BlockSpec((BN, D), lambda i: (i, 0)),
        out_shape=jax.ShapeDtypeStruct((N, D), jnp.float32),
    )(part, deg, h, ws, bs, wn, w2, b2)


def kernel(x, edge_index, W_d1, b_d1, W_self, b_self, W_neigh, W_d2, b_d2):
    h = _d1(x, W_d1, b_d1.reshape(1, D))
    idx = jnp.transpose(edge_index.reshape(2, NW, NCHUNK, CH), (1, 2, 0, 3))
    zeros = jnp.zeros((RPS, D), jnp.float32)
    part, deg = _sc_agg(h, idx, zeros)
    return _out(part, deg, h, W_self,
                b_self.reshape(1, D), W_neigh, W_d2, b_d2.reshape(1, D))


# speedup vs baseline: 14.8212x; 1.0717x over previous
import functools

import jax
import jax.numpy as jnp
from jax import lax
from jax.experimental import pallas as pl
from jax.experimental.pallas import tpu as pltpu
from jax.experimental.pallas import tpu_sc as plsc

N = 10000
E = 320000
D = 128
NC, NS = 2, 16
NW = NC * NS
EPW = E // NW
CH = 80
NCHUNK = EPW // CH
NBLK = 15
NBLKS = NCHUNK // NBLK
NCP = (NBLKS + 1) * NBLK
NP = 10240
RPS = NP // NS
L = 16

BN = 1024


def _d1_body(x_ref, w_ref, b_ref, out_ref):
    h = jnp.dot(x_ref[...], w_ref[...], preferred_element_type=jnp.float32)
    out_ref[...] = jnp.maximum(h + b_ref[...], 0.0)


def _d1(x, w, b):
    return pl.pallas_call(
        _d1_body,
        grid=(NP // BN,),
        in_specs=[
            pl.BlockSpec((BN, D), lambda i: (i, 0)),
            pl.BlockSpec((D, D), lambda i: (0, 0)),
            pl.BlockSpec((1, D), lambda i: (0, 0)),
        ],
        out_specs=pl.BlockSpec((BN, D), lambda i: (i, 0)),
        out_shape=jax.ShapeDtypeStruct((N, D), jnp.float32),
    )(x, w, b)


_MESH = plsc.VectorSubcoreMesh(
    core_axis_name="c", subcore_axis_name="s", num_cores=NC, num_subcores=NS)


@functools.partial(
    pl.kernel,
    out_type=(jax.ShapeDtypeStruct((NC, NP, D), jnp.float32),
              jax.ShapeDtypeStruct((NW, N), jnp.float32)),
    mesh=_MESH,
    compiler_params=pltpu.CompilerParams(needs_layout_passes=False),
    scratch_types=[
        pltpu.VMEM((NBLK, 2, CH), jnp.int32),
        pltpu.VMEM((NBLK, 2, CH), jnp.int32),
        pltpu.VMEM((N,), jnp.float32),
        pltpu.VMEM((CH, D), jnp.float32),
        pltpu.VMEM((CH, D), jnp.float32),
        pltpu.VMEM((CH, D), jnp.float32),
        pltpu.VMEM_SHARED((NP, D), jnp.float32),
        pltpu.SemaphoreType.DMA,
        pltpu.SemaphoreType.DMA,
        pltpu.SemaphoreType.DMA,
        pltpu.SemaphoreType.DMA,
        pltpu.SemaphoreType.DMA,
    ],
)
def _sc_agg(h_hbm, idx_hbm, out_hbm, deg_hbm,
            tib_a, tib_b, hist_v, r0, r1, r2, agg_sh,
            g0, g1, g2, isem_a, isem_b):
    cid = lax.axis_index("c")
    sid = lax.axis_index("s")
    wid = cid * NS + sid
    rbufs = (r0, r1, r2)
    gsems = (g0, g1, g2)

    def _blk_load(b, tib, isem):
        pltpu.async_copy(idx_hbm.at[wid, pl.ds(NBLK * b, NBLK)], tib, isem)

    def _blk_wait(tib, isem):
        pltpu.make_async_copy(idx_hbm.at[wid, pl.ds(0, NBLK)], tib,
                              isem).wait()

    def _gather(idx_row, buf, sem):
        pltpu.async_copy(h_hbm.at[idx_row], buf, sem)

    def _gwait(buf, sem):
        pltpu.make_async_copy(h_hbm.at[tib_a.at[0, 0]], buf, sem).wait()

    def _scat(buf, idx_row):
        pltpu.sync_copy(buf, agg_sh.at[idx_row], add=True)

    ones = jnp.ones((L,), jnp.float32)

    def _hist(tib, k):
        for j in range(CH // L):
            d16 = tib[k, 1, pl.ds(j * L, L)]
            plsc.addupdate_scatter(hist_v, [d16], ones)

    _blk_load(0, tib_a, isem_a)
    _blk_load(1, tib_b, isem_b)
    zeros16 = jnp.zeros((L,), jnp.float32)

    def _zero_r0(j, carry):
        r0[j % CH, pl.ds((j // CH) * L, L)] = zeros16
        return carry

    lax.fori_loop(0, CH * (D // L), _zero_r0, 0)
    for i in range(RPS // CH):
        pltpu.sync_copy(r0, agg_sh.at[pl.ds(sid * RPS + i * CH, CH)])

    def _zero_hist(j, carry):
        hist_v[pl.ds(j * L, L)] = zeros16
        return carry

    lax.fori_loop(0, N // L, _zero_hist, 0)
    _blk_wait(tib_a, isem_a)
    for k in range(3):
        _gather(tib_a.at[k, 0], rbufs[k], gsems[k])
    plsc.subcore_barrier()

    def _half(b, cur, nxt, isem_cur, isem_nxt):
        _blk_wait(nxt, isem_nxt)
        for m in range(NBLK):
            k = m % 3
            _gwait(rbufs[k], gsems[k])
            _scat(rbufs[k], cur.at[m, 1])
            if m < NBLK - 3:
                _gather(cur.at[m + 3, 0], rbufs[k], gsems[k])
            else:
                _gather(nxt.at[m + 3 - NBLK, 0], rbufs[k], gsems[k])
            _hist(cur, m)
        _blk_load(jnp.minimum(b + 2, NBLKS), cur, isem_cur)

    def _pair(i, carry):
        b = 2 * i
        _half(b, tib_a, tib_b, isem_a, isem_b)
        _half(b + 1, tib_b, tib_a, isem_b, isem_a)
        return carry

    lax.fori_loop(0, NBLKS // 2, _pair, 0)
    _blk_wait(tib_b, isem_b)
    for m in range(5):
        k = m % 3
        _gwait(rbufs[k], gsems[k])
        _scat(rbufs[k], tib_a.at[m, 1])
        if m + 3 < 5:
            _gather(tib_a.at[m + 3, 0], rbufs[k], gsems[k])
        _hist(tib_a, m)

    pltpu.sync_copy(hist_v, deg_hbm.at[wid])
    plsc.subcore_barrier()
    pltpu.sync_copy(agg_sh.at[pl.ds(sid * RPS, RPS)],
                    out_hbm.at[cid, pl.ds(sid * RPS, RPS)])


def _out_body(part_ref, deg_ref, h_ref, ws_ref, bs_ref, wn_ref, w2_ref,
              b2_ref, o_ref):
    agg = part_ref[0] + part_ref[1]
    degt = jnp.transpose(deg_ref[...])
    deg = jnp.maximum(jnp.sum(degt, axis=1, keepdims=True), 1.0)
    h_neigh = agg / deg
    h = h_ref[...]
    h2 = jnp.dot(h, ws_ref[...], preferred_element_type=jnp.float32)
    h2 = h2 + jnp.dot(h_neigh, wn_ref[...], preferred_element_type=jnp.float32)
    h2 = jnp.maximum(h2 + bs_ref[...], 0.0)
    o = jnp.dot(h2, w2_ref[...], preferred_element_type=jnp.float32)
    o_ref[...] = jnp.maximum(o + b2_ref[...], 0.0)


def _out(part, deg, h, ws, bs, wn, w2, b2):
    return pl.pallas_call(
        _out_body,
        grid=(NP // BN,),
        in_specs=[
            pl.BlockSpec((NC, BN, D), lambda i: (0, i, 0)),
            pl.BlockSpec((NW, BN), lambda i: (0, i)),
            pl.BlockSpec((BN, D), lambda i: (i, 0)),
            pl.BlockSpec((D, D), lambda i: (0, 0)),
            pl.BlockSpec((1, D), lambda i: (0, 0)),
            pl.BlockSpec((D, D), lambda i: (0, 0)),
            pl.BlockSpec((D, D), lambda i: (0, 0)),
            pl.BlockSpec((1, D), lambda i: (0, 0)),
        ],
        out_specs=pl.BlockSpec((BN, D), lambda i: (i, 0)),
        out_shape=jax.ShapeDtypeStruct((N, D), jnp.float32),
    )(part, deg, h, ws, bs, wn, w2, b2)


def kernel(x, edge_index, W_d1, b_d1, W_self, b_self, W_neigh, W_d2, b_d2):
    h = _d1(x, W_d1, b_d1.reshape(1, D))
    idx = jnp.transpose(edge_index.reshape(2, NW, NCHUNK, CH), (1, 2, 0, 3))
    idx = jnp.pad(idx, ((0, 0), (0, NCP - NCHUNK), (0, 0), (0, 0)))
    part, deg = _sc_agg(h, idx)
    return _out(part, deg, h, W_self,
                b_self.reshape(1, D), W_neigh, W_d2, b_d2.reshape(1, D))
